# baseline (speedup 1.0000x reference)
import functools

import jax
import jax.numpy as jnp
from jax import lax
from jax.experimental import pallas as pl
from jax.experimental.pallas import tpu as pltpu
from jax.experimental.pallas import tpu_sc as plsc

_N_ROWS = 16384
_N_COLS = 1024
_NC = 2
_NS = 16
_GROUP = 16
_ROWS_PER_WORKER = _N_ROWS // (_NC * _NS)
_GROUPS_PER_WORKER = _ROWS_PER_WORKER // _GROUP
_INTERPRET = False


def _sc_scan(x_hbm, out_hbm, xbuf, obuf):
    wid = lax.axis_index("s") * _NC + lax.axis_index("c")
    base = wid * _ROWS_PER_WORKER
    row_idx = lax.iota(jnp.int32, _GROUP)

    def group_body(g, carry_unused):
        row0 = base + g * _GROUP
        pltpu.sync_copy(x_hbm.at[pl.ds(row0, _GROUP)], xbuf)

        def col_body(j, acc):
            col = jnp.full((_GROUP,), j, dtype=jnp.int32)
            plsc.store_scatter(obuf, [row_idx, col], acc)
            v = plsc.load_gather(xbuf, [row_idx, col])
            return acc + v

        lax.fori_loop(0, _N_COLS, col_body, jnp.zeros((_GROUP,), jnp.float32))
        pltpu.sync_copy(obuf, out_hbm.at[pl.ds(row0, _GROUP)])
        return carry_unused

    lax.fori_loop(0, _GROUPS_PER_WORKER, group_body, jnp.int32(0))


def kernel(x):
    fn = functools.partial(
        pl.kernel,
        mesh=plsc.VectorSubcoreMesh(core_axis_name="c", subcore_axis_name="s"),
        out_type=jax.ShapeDtypeStruct((_N_ROWS, _N_COLS), jnp.float32),
        scratch_types=[
            pltpu.VMEM((_GROUP, _N_COLS), jnp.float32),
            pltpu.VMEM((_GROUP, _N_COLS), jnp.float32),
        ],
        compiler_params=pltpu.CompilerParams(use_tc_tiling_on_sc=False, needs_layout_passes=False),
        interpret=_INTERPRET,
    )(_sc_scan)
    return fn(x)


# speedup vs baseline: 18.5541x; 18.5541x over previous
import jax
import jax.numpy as jnp
from jax.experimental import pallas as pl
from jax.experimental.pallas import tpu as pltpu


_BLOCK_ROWS = 2048
_CHUNK = 128


def _scan_kernel(x_ref, o_ref):
    x = x_ref[...]
    rows, n = x.shape
    c = _CHUNK
    nchunk = n // c
    f32 = jnp.float32

    rr = jax.lax.broadcasted_iota(jnp.int32, (c, c), 0)
    cc = jax.lax.broadcasted_iota(jnp.int32, (c, c), 1)
    texc = (rr < cc).astype(f32)

    carry = jnp.zeros((rows, 1), dtype=f32)
    for k in range(nchunk):
        xk = x[:, k * c : (k + 1) * c]
        part = jnp.dot(xk, texc, preferred_element_type=f32)
        o_ref[:, k * c : (k + 1) * c] = part + jnp.broadcast_to(carry, (rows, c))
        if k + 1 < nchunk:
            carry = carry + part[:, c - 1 : c] + xk[:, c - 1 : c]


def kernel(x):
    n_rows, n = x.shape
    grid = (n_rows // _BLOCK_ROWS,)
    return pl.pallas_call(
        _scan_kernel,
        grid=grid,
        in_specs=[pl.BlockSpec((_BLOCK_ROWS, n), lambda i: (i, 0))],
        out_specs=pl.BlockSpec((_BLOCK_ROWS, n), lambda i: (i, 0)),
        out_shape=jax.ShapeDtypeStruct((n_rows, n), x.dtype),
        compiler_params=pltpu.CompilerParams(
            dimension_semantics=("parallel",)
        ),
    )(x)
